# Initial kernel scaffold; baseline (speedup 1.0000x reference)
import jax
import jax.numpy as jnp
from jax.experimental import pallas as pl


def kernel(x, edge_index, batch, W1, b1, W2, b2, Wl1, bl1, Wl2, bl2, Wl3, bl3):
    raise NotImplementedError("write your pallas kernel here")


# speedup vs baseline: 16.0101x; 16.0101x over previous
import functools

import jax
import jax.numpy as jnp
from jax import lax
from jax.experimental import pallas as pl
from jax.experimental.pallas import tpu as pltpu
from jax.experimental.pallas import tpu_sc as plsc

N = 10000
E = 320000
D = 128
G = 64
NC = 2
NS = 16
LANES = 16
NW = NC * NS
EB = 128
NBLK = E // EB
RPT = 624
RPT_LAST = N - (NS - 1) * RPT
BLK = 1000
NBK = N // BLK

_HIGH = lax.Precision.HIGHEST
_mesh = plsc.VectorSubcoreMesh(core_axis_name="c", subcore_axis_name="s")


def _tile_chunks(k):
    base = k * RPT
    if k < NS - 1:
        return [(base, EB), (base + EB, EB), (base + 2 * EB, EB),
                (base + 3 * EB, EB), (base + 4 * EB, RPT - 4 * EB)]
    return [(base + i * EB, EB) for i in range(5)]


@functools.partial(
    pl.kernel,
    mesh=_mesh,
    out_type=jax.ShapeDtypeStruct((NC, N, LANES), jnp.float32),
    scratch_types=[
        pltpu.VMEM((EB,), jnp.int32),
        pltpu.VMEM((EB, LANES), jnp.float32),
        pltpu.VMEM_SHARED((N, LANES), jnp.float32),
    ],
)
def _deg_sc(dst_hbm, out_hbm, idx_v, buf_v, acc):
    c = lax.axis_index("c")
    s = lax.axis_index("s")
    wid = s * NC + c

    @pl.loop(0, EB)
    def _(i):
        buf_v[i, :] = jnp.zeros((LANES,), jnp.float32)

    for k in range(NS):
        @pl.when(s == k)
        def _():
            for off, sz in _tile_chunks(k):
                pltpu.sync_copy(buf_v.at[pl.ds(0, sz)], acc.at[pl.ds(off, sz)])

    @pl.loop(0, EB)
    def _(i):
        buf_v[i, :] = jnp.ones((LANES,), jnp.float32)

    plsc.subcore_barrier()

    @pl.loop(wid, NBLK, step=NW)
    def _(b):
        pltpu.sync_copy(dst_hbm.at[pl.ds(b * EB, EB)], idx_v)
        pltpu.sync_copy(buf_v, acc.at[idx_v], add=True)

    plsc.subcore_barrier()

    for k in range(NS):
        @pl.when(s == k)
        def _():
            for off, sz in _tile_chunks(k):
                pltpu.sync_copy(acc.at[pl.ds(off, sz)], buf_v.at[pl.ds(0, sz)])
                pltpu.sync_copy(buf_v.at[pl.ds(0, sz)],
                                out_hbm.at[c, pl.ds(off, sz)])


@functools.partial(
    pl.kernel,
    mesh=_mesh,
    out_type=jax.ShapeDtypeStruct((NC, N, D), jnp.float32),
    scratch_types=[
        pltpu.VMEM((EB,), jnp.int32),
        pltpu.VMEM((EB,), jnp.int32),
        pltpu.VMEM((EB, D), jnp.float32),
        pltpu.VMEM_SHARED((N, D), jnp.float32),
        pltpu.SemaphoreType.DMA,
    ],
)
def _msg_sc(hd_hbm, src_hbm, dst_hbm, out_hbm, sidx, didx, rows, acc, sem):
    c = lax.axis_index("c")
    s = lax.axis_index("s")
    wid = s * NC + c

    @pl.loop(0, EB)
    def _(i):
        @pl.loop(0, D, step=LANES)
        def _(j):
            rows[i, pl.ds(j, LANES)] = jnp.zeros((LANES,), jnp.float32)

    for k in range(NS):
        @pl.when(s == k)
        def _():
            for off, sz in _tile_chunks(k):
                pltpu.sync_copy(rows.at[pl.ds(0, sz)], acc.at[pl.ds(off, sz)])

    plsc.subcore_barrier()

    @pl.loop(wid, NBLK, step=NW)
    def _(b):
        pltpu.sync_copy(src_hbm.at[pl.ds(b * EB, EB)], sidx)
        pltpu.async_copy(hd_hbm.at[sidx], rows, sem).wait()
        pltpu.sync_copy(dst_hbm.at[pl.ds(b * EB, EB)], didx)
        pltpu.sync_copy(rows, acc.at[didx], add=True)

    plsc.subcore_barrier()

    for k in range(NS):
        @pl.when(s == k)
        def _():
            for off, sz in _tile_chunks(k):
                pltpu.sync_copy(acc.at[pl.ds(off, sz)], rows.at[pl.ds(0, sz)])
                pltpu.sync_copy(rows.at[pl.ds(0, sz)],
                                out_hbm.at[c, pl.ds(off, sz)])


def _mm_body(x_ref, w_ref, o_ref):
    o_ref[...] = jnp.dot(x_ref[...], w_ref[...],
                         preferred_element_type=jnp.float32, precision=_HIGH)


def _k0(x, W1):
    return pl.pallas_call(
        _mm_body,
        grid=(NBK,),
        in_specs=[pl.BlockSpec((BLK, D), lambda i: (i, 0)),
                  pl.BlockSpec((D, D), lambda i: (0, 0))],
        out_specs=pl.BlockSpec((BLK, D), lambda i: (i, 0)),
        out_shape=jax.ShapeDtypeStruct((N, D), jnp.float32),
    )(x, W1)


def _k1_body(degp_ref, xw_ref, dinv_ref, hd_ref):
    deg = degp_ref[0, :, 0:1] + degp_ref[1, :, 0:1] + 1.0
    dinv = lax.rsqrt(deg)
    dinv_ref[...] = dinv
    hd_ref[...] = xw_ref[...] * dinv


def _k1(degp, xw):
    return pl.pallas_call(
        _k1_body,
        grid=(NBK,),
        in_specs=[pl.BlockSpec((NC, BLK, LANES), lambda i: (0, i, 0)),
                  pl.BlockSpec((BLK, D), lambda i: (i, 0))],
        out_specs=[pl.BlockSpec((BLK, 1), lambda i: (i, 0)),
                   pl.BlockSpec((BLK, D), lambda i: (i, 0))],
        out_shape=[jax.ShapeDtypeStruct((N, 1), jnp.float32),
                   jax.ShapeDtypeStruct((N, D), jnp.float32)],
    )(degp, xw)


def _k2_body(p_ref, hd_ref, dinv_ref, b_ref, w_ref, o_ref):
    agg = p_ref[0] + p_ref[1] + hd_ref[...]
    h1 = jnp.maximum(dinv_ref[...] * agg + b_ref[...], 0.0)
    o_ref[...] = jnp.dot(h1, w_ref[...], preferred_element_type=jnp.float32,
                         precision=_HIGH) * dinv_ref[...]


def _k2(p, hd, dinv, b1, W2):
    return pl.pallas_call(
        _k2_body,
        grid=(NBK,),
        in_specs=[pl.BlockSpec((NC, BLK, D), lambda i: (0, i, 0)),
                  pl.BlockSpec((BLK, D), lambda i: (i, 0)),
                  pl.BlockSpec((BLK, 1), lambda i: (i, 0)),
                  pl.BlockSpec((1, D), lambda i: (0, 0)),
                  pl.BlockSpec((D, D), lambda i: (0, 0))],
        out_specs=pl.BlockSpec((BLK, D), lambda i: (i, 0)),
        out_shape=jax.ShapeDtypeStruct((N, D), jnp.float32),
    )(p, hd, dinv, b1, W2)


def _k3_body(p_ref, hd_ref, dinv_ref, b_ref, batch_ref,
             wl1_ref, bl1_ref, wl2_ref, bl2_ref, wl3_ref, bl3_ref,
             o_ref, sum_acc, cnt_acc):
    i = pl.program_id(0)

    @pl.when(i == 0)
    def _():
        sum_acc[...] = jnp.zeros_like(sum_acc)
        cnt_acc[...] = jnp.zeros_like(cnt_acc)

    agg = p_ref[0] + p_ref[1] + hd_ref[...]
    h2 = jnp.maximum(dinv_ref[...] * agg + b_ref[...], 0.0)
    gids = lax.broadcasted_iota(jnp.int32, (G, BLK), 0)
    onehot = (batch_ref[0] == gids).astype(jnp.float32)
    sum_acc[...] += jnp.dot(onehot, h2, preferred_element_type=jnp.float32,
                            precision=_HIGH)
    cnt_acc[...] += jnp.sum(onehot, axis=1, keepdims=True)

    @pl.when(i == NBK - 1)
    def _():
        g = sum_acc[...] / jnp.maximum(cnt_acc[...], 1.0)
        g = jnp.maximum(jnp.dot(g, wl1_ref[...],
                                preferred_element_type=jnp.float32,
                                precision=_HIGH) + bl1_ref[...], 0.0)
        g = jnp.maximum(jnp.dot(g, wl2_ref[...],
                                preferred_element_type=jnp.float32,
                                precision=_HIGH) + bl2_ref[...], 0.0)
        o_ref[...] = (jnp.sum(g * wl3_ref[...], axis=1, keepdims=True)
                      + bl3_ref[...])


def _k3(p, hd, dinv, b2, batch3, Wl1, bl1, Wl2, bl2, Wl3t, bl3):
    return pl.pallas_call(
        _k3_body,
        grid=(NBK,),
        in_specs=[pl.BlockSpec((NC, BLK, D), lambda i: (0, i, 0)),
                  pl.BlockSpec((BLK, D), lambda i: (i, 0)),
                  pl.BlockSpec((BLK, 1), lambda i: (i, 0)),
                  pl.BlockSpec((1, D), lambda i: (0, 0)),
                  pl.BlockSpec((1, 1, BLK), lambda i: (i, 0, 0)),
                  pl.BlockSpec((D, G), lambda i: (0, 0)),
                  pl.BlockSpec((1, G), lambda i: (0, 0)),
                  pl.BlockSpec((G, 32), lambda i: (0, 0)),
                  pl.BlockSpec((1, 32), lambda i: (0, 0)),
                  pl.BlockSpec((1, 32), lambda i: (0, 0)),
                  pl.BlockSpec((1, 1), lambda i: (0, 0))],
        out_specs=pl.BlockSpec((G, 1), lambda i: (0, 0)),
        out_shape=jax.ShapeDtypeStruct((G, 1), jnp.float32),
        scratch_shapes=[pltpu.VMEM((G, D), jnp.float32),
                        pltpu.VMEM((G, 1), jnp.float32)],
    )(p, hd, dinv, b2, batch3, Wl1, bl1, Wl2, bl2, Wl3t, bl3)


def kernel(x, edge_index, batch, W1, b1, W2, b2, Wl1, bl1, Wl2, bl2, Wl3, bl3):
    src = edge_index[0]
    dst = edge_index[1]

    degp = _deg_sc(dst)
    xw1 = _k0(x, W1)
    dinv, hd1 = _k1(degp, xw1)

    p1 = _msg_sc(hd1, src, dst)
    hd2 = _k2(p1, hd1, dinv, b1.reshape(1, D), W2)

    p2 = _msg_sc(hd2, src, dst)
    out = _k3(p2, hd2, dinv, b2.reshape(1, D), batch.reshape(NBK, 1, BLK),
              Wl1, bl1.reshape(1, G), Wl2, bl2.reshape(1, 32),
              Wl3.reshape(1, 32), bl3.reshape(1, 1))
    return out
